# baseline (speedup 1.0000x reference)
import functools

import jax
import jax.numpy as jnp
from jax import lax
from jax.experimental import pallas as pl
from jax.experimental.pallas import tpu as pltpu
from jax.experimental.pallas import tpu_sc as plsc

_N = 10000
_NPAD = 10240
_D = 128
_NG = 64
_E = 320000
_NT = 400
_NC = 2
_NS = 16
_CHUNK = 128
_CH = 80
_EPAD = _NC * _NS * _CH * _CHUNK
_R = 1024
_NB = _NPAD // _R
_ROWS_PER_SUB = _NPAD // _NS
_PREC = lax.Precision.HIGHEST


def _sc_scatter_add(u, src3, dst3, zrows):
    mesh = plsc.VectorSubcoreMesh(core_axis_name="c", subcore_axis_name="s")

    @functools.partial(
        pl.kernel,
        mesh=mesh,
        out_type=jax.ShapeDtypeStruct((_NC, _NPAD, _D), jnp.float32),
        scratch_types=[
            pltpu.VMEM_SHARED((_NPAD, _D), jnp.float32),
            pltpu.VMEM((_CHUNK,), jnp.int32),
            pltpu.VMEM((_CHUNK,), jnp.int32),
            pltpu.VMEM((_CHUNK, _D), jnp.float32),
            pltpu.SemaphoreType.DMA,
        ],
    )
    def k(u_hbm, src_hbm, dst_hbm, z_hbm, out_hbm, agg_sh, sidx, didx, rows, sem):
        c = lax.axis_index("c")
        s = lax.axis_index("s")
        base = s * _ROWS_PER_SUB
        pltpu.sync_copy(z_hbm, agg_sh.at[pl.ds(base, _ROWS_PER_SUB)])
        plsc.subcore_barrier()

        @pl.loop(0, _CH)
        def _(kk):
            pltpu.sync_copy(src_hbm.at[c, s, kk], sidx)
            pltpu.sync_copy(dst_hbm.at[c, s, kk], didx)
            pltpu.async_copy(u_hbm.at[sidx], rows, sem).wait()
            pltpu.sync_copy(rows, agg_sh.at[didx], add=True)

        plsc.subcore_barrier()
        pltpu.sync_copy(agg_sh.at[pl.ds(base, _ROWS_PER_SUB)],
                        out_hbm.at[c, pl.ds(base, _ROWS_PER_SUB)])

    return k(u, src3, dst3, zrows)


def _rowmask(i, v):
    rid = i * _R + lax.broadcasted_iota(jnp.int32, (_R, 1), 0)
    return jnp.where(rid < _N, v, 0.0)


def _dot(a, b):
    return jnp.dot(a, b, precision=_PREC, preferred_element_type=jnp.float32)


def _proj_body(x_ref, emb_ref, w1a_ref, w1b_ref, u_ref):
    i = pl.program_id(0)
    x = x_ref[...]
    table2 = _dot(emb_ref[...], w1b_ref[...])
    types = x[:, 0:1].astype(jnp.int32)
    iota = lax.broadcasted_iota(jnp.int32, (_R, _NT), 1)
    onehot = (types == iota).astype(jnp.float32)
    u = _dot(x, w1a_ref[...]) + _dot(onehot, table2)
    u_ref[...] = _rowmask(i, u)


def _ln(t, g, b):
    mu = jnp.mean(t, axis=-1, keepdims=True)
    var = jnp.mean((t - mu) ** 2, axis=-1, keepdims=True)
    return (t - mu) / jnp.sqrt(var + 1e-5) * g + b


def _conv_body(u_ref, a_ref, b1_ref, w2_ref, b2_ref, g_ref, bb_ref, w1n_ref,
               o_ref):
    i = pl.program_id(0)
    z = jnp.maximum(u_ref[...] + a_ref[0] + a_ref[1] + b1_ref[...], 0.0)
    t = _dot(z, w2_ref[...]) + b2_ref[...]
    h = jnp.maximum(_ln(t, g_ref[...], bb_ref[...]), 0.0)
    o_ref[...] = _rowmask(i, _dot(h, w1n_ref[...]))


def _conv_gate_body(u_ref, a_ref, batch_ref, b1_ref, w2_ref, b2_ref, g_ref, bb_ref,
                    gw1_ref, gb1_ref, gw2_ref, gb2_ref,
                    h_out_ref, gate_out_ref, gmax_out_ref, gmax_acc):
    i = pl.program_id(0)
    nb = pl.num_programs(0)
    z = jnp.maximum(u_ref[...] + a_ref[0] + a_ref[1] + b1_ref[...], 0.0)
    t = _dot(z, w2_ref[...]) + b2_ref[...]
    h = jnp.maximum(_ln(t, g_ref[...], bb_ref[...]), 0.0)
    h = _rowmask(i, h)
    h_out_ref[...] = h
    g1 = jnp.maximum(_dot(h, gw1_ref[...]) + gb1_ref[...], 0.0)
    gate = _rowmask(i, _dot(g1, gw2_ref[...]) + gb2_ref[...])
    gate_out_ref[...] = gate
    m = batch_ref[...] == lax.broadcasted_iota(jnp.int32, (_R, _NG), 1)
    smax = jnp.max(jnp.where(m, gate, -1e30), axis=0, keepdims=True)

    @pl.when(i == 0)
    def _():
        gmax_acc[...] = jnp.full((8, _NG), -1e30, jnp.float32)

    gmax_acc[...] = jnp.maximum(gmax_acc[...], jnp.broadcast_to(smax, (8, _NG)))

    @pl.when(i == nb - 1)
    def _():
        gmax_out_ref[...] = gmax_acc[...]


def _pool_body(h_ref, gate_ref, gmax_ref, batch_ref, cw1_ref, cb1_ref, cw2_ref,
               cb2_ref, out_ref, num_acc, den_acc):
    i = pl.program_id(0)
    nb = pl.num_programs(0)

    @pl.when(i == 0)
    def _():
        num_acc[...] = jnp.zeros_like(num_acc)
        den_acc[...] = jnp.zeros_like(den_acc)

    m = (batch_ref[...] == lax.broadcasted_iota(jnp.int32, (_R, _NG), 1)
         ).astype(jnp.float32)
    gmax_i = jnp.sum(m * gmax_ref[0:1, :], axis=1, keepdims=True)
    alpha = jnp.exp(gate_ref[...] - gmax_i)
    wh = alpha * h_ref[...]
    cdims = (((0,), (0,)), ((), ()))
    num_acc[...] += lax.dot_general(m, wh, cdims, precision=_PREC,
                                    preferred_element_type=jnp.float32)
    den_acc[...] += lax.dot_general(m, alpha, cdims, precision=_PREC,
                                    preferred_element_type=jnp.float32)

    @pl.when(i == nb - 1)
    def _():
        pooled = num_acc[...] / (den_acc[...] + 1e-16)
        p1 = jnp.maximum(_dot(pooled, cw1_ref[...]) + cb1_ref[...], 0.0)
        out_ref[...] = _dot(p1, cw2_ref[...]) + cb2_ref[...]


def _row_spec(shape):
    nd = len(shape)
    return pl.BlockSpec(shape, lambda i: (i,) + (0,) * (nd - 1))


def _full_spec(shape):
    nd = len(shape)
    return pl.BlockSpec(shape, lambda i: (0,) * nd)


def kernel(x, edge_index, batch, emb, conv0_W1, conv0_b1, conv0_W2, conv0_b2,
           conv1_W1, conv1_b1, conv1_W2, conv1_b2, ln0_g, ln0_b, ln1_g, ln1_b,
           gate_W1, gate_b1, gate_W2, gate_b2, cls_W1, cls_b1, cls_W2, cls_b2):
    f32 = jnp.float32
    xp = jnp.pad(x, ((0, _NPAD - _N), (0, 0)))
    batch_p = jnp.pad(batch, (0, _NPAD - _N), constant_values=_NG).reshape(_NPAD, 1)
    src3 = jnp.pad(edge_index[0], (0, _EPAD - _E),
                   constant_values=_N).reshape(_NC, _NS, _CH, _CHUNK)
    dst3 = jnp.pad(edge_index[1], (0, _EPAD - _E),
                   constant_values=_N).reshape(_NC, _NS, _CH, _CHUNK)
    zrows = jnp.zeros((_ROWS_PER_SUB, _D), f32)
    w1a = jnp.pad(conv0_W1[:_D - 1], ((1, 0), (0, 0)))
    w1b = conv0_W1[_D - 1:]
    r2 = lambda v: v.reshape(1, -1)

    u0 = pl.pallas_call(
        _proj_body,
        grid=(_NB,),
        in_specs=[_row_spec((_R, _D)), _full_spec((_NT, 16)),
                  _full_spec((_D, _D)), _full_spec((16, _D))],
        out_specs=_row_spec((_R, _D)),
        out_shape=jax.ShapeDtypeStruct((_NPAD, _D), f32),
    )(xp, emb, w1a, w1b)

    agg0 = _sc_scatter_add(u0, src3, dst3, zrows)

    u1 = pl.pallas_call(
        _conv_body,
        grid=(_NB,),
        in_specs=[_row_spec((_R, _D)),
                  pl.BlockSpec((_NC, _R, _D), lambda i: (0, i, 0)),
                  _full_spec((1, _D)), _full_spec((_D, _D)), _full_spec((1, _D)),
                  _full_spec((1, _D)), _full_spec((1, _D)), _full_spec((_D, _D))],
        out_specs=_row_spec((_R, _D)),
        out_shape=jax.ShapeDtypeStruct((_NPAD, _D), f32),
    )(u0, agg0, r2(conv0_b1), conv0_W2, r2(conv0_b2), r2(ln0_g), r2(ln0_b),
      conv1_W1)

    agg1 = _sc_scatter_add(u1, src3, dst3, zrows)

    h2, gate, gmax = pl.pallas_call(
        _conv_gate_body,
        grid=(_NB,),
        in_specs=[_row_spec((_R, _D)),
                  pl.BlockSpec((_NC, _R, _D), lambda i: (0, i, 0)),
                  _row_spec((_R, 1)),
                  _full_spec((1, _D)), _full_spec((_D, _D)), _full_spec((1, _D)),
                  _full_spec((1, _D)), _full_spec((1, _D)),
                  _full_spec((_D, _D)), _full_spec((1, _D)),
                  _full_spec((_D, 1)), _full_spec((1, 1))],
        out_specs=[_row_spec((_R, _D)), _row_spec((_R, 1)), _full_spec((8, _NG))],
        out_shape=[jax.ShapeDtypeStruct((_NPAD, _D), f32),
                   jax.ShapeDtypeStruct((_NPAD, 1), f32),
                   jax.ShapeDtypeStruct((8, _NG), f32)],
        scratch_shapes=[pltpu.VMEM((8, _NG), f32)],
    )(u1, agg1, batch_p, r2(conv1_b1), conv1_W2, r2(conv1_b2), r2(ln1_g),
      r2(ln1_b), gate_W1, r2(gate_b1), gate_W2, gate_b2.reshape(1, 1))

    out = pl.pallas_call(
        _pool_body,
        grid=(_NB,),
        in_specs=[_row_spec((_R, _D)), _row_spec((_R, 1)), _full_spec((8, _NG)),
                  _row_spec((_R, 1)),
                  _full_spec((_D, _D)), _full_spec((1, _D)),
                  _full_spec((_D, 2)), _full_spec((1, 2))],
        out_specs=_full_spec((_NG, 2)),
        out_shape=jax.ShapeDtypeStruct((_NG, 2), f32),
        scratch_shapes=[pltpu.VMEM((_NG, _D), f32), pltpu.VMEM((_NG, 1), f32)],
    )(h2, gate, gmax, batch_p, cls_W1, r2(cls_b1), cls_W2, cls_b2.reshape(1, 2))

    return out


# speedup vs baseline: 2.0081x; 2.0081x over previous
import functools

import jax
import jax.numpy as jnp
from jax import lax
from jax.experimental import pallas as pl
from jax.experimental.pallas import tpu as pltpu
from jax.experimental.pallas import tpu_sc as plsc

_N = 10000
_NPAD = 10240
_D = 128
_NG = 64
_E = 320000
_NT = 400
_NC = 2
_NS = 16
_CHUNK = 128
_CH = 80
_EPAD = _NC * _NS * _CH * _CHUNK
_R = 1024
_NB = _NPAD // _R
_ROWS_PER_SUB = _NPAD // _NS
_PREC = lax.Precision.HIGHEST


def _sc_scatter_add(u, src3, dst3, zrows):
    mesh = plsc.VectorSubcoreMesh(core_axis_name="c", subcore_axis_name="s")

    @functools.partial(
        pl.kernel,
        mesh=mesh,
        out_type=jax.ShapeDtypeStruct((_NC, _NPAD, _D), jnp.float32),
        scratch_types=[
            pltpu.VMEM_SHARED((_NPAD, _D), jnp.float32),
            pltpu.VMEM((_CHUNK,), jnp.int32),
            pltpu.VMEM((_CHUNK,), jnp.int32),
            pltpu.VMEM((_CHUNK, _D), jnp.float32),
            pltpu.SemaphoreType.DMA,
        ],
    )
    def k(u_hbm, src_hbm, dst_hbm, z_hbm, out_hbm, agg_sh, sidx, didx, rows, sem):
        c = lax.axis_index("c")
        s = lax.axis_index("s")
        base = s * _ROWS_PER_SUB
        pltpu.sync_copy(z_hbm, agg_sh.at[pl.ds(base, _ROWS_PER_SUB)])
        plsc.subcore_barrier()

        @pl.loop(0, _CH)
        def _(kk):
            pltpu.sync_copy(src_hbm.at[c, s, kk], sidx)
            pltpu.sync_copy(dst_hbm.at[c, s, kk], didx)
            pltpu.async_copy(u_hbm.at[sidx], rows, sem).wait()
            pltpu.sync_copy(rows, agg_sh.at[didx], add=True)

        plsc.subcore_barrier()
        pltpu.sync_copy(agg_sh.at[pl.ds(base, _ROWS_PER_SUB)],
                        out_hbm.at[c, pl.ds(base, _ROWS_PER_SUB)])

    return k(u, src3, dst3, zrows)


def _rowmask(i, v):
    rid = i * _R + lax.broadcasted_iota(jnp.int32, (_R, 1), 0)
    return jnp.where(rid < _N, v, 0.0)


def _dot(a, b):
    return jnp.dot(a, b, precision=_PREC, preferred_element_type=jnp.float32)


def _proj_body(x_ref, emb_ref, w1a_ref, w1b_ref, u_ref):
    i = pl.program_id(0)
    x = x_ref[...]
    table2 = _dot(emb_ref[...], w1b_ref[...])
    types = x[:, 0:1].astype(jnp.int32)
    iota = lax.broadcasted_iota(jnp.int32, (_R, _NT), 1)
    onehot = (types == iota).astype(jnp.float32)
    u = _dot(x, w1a_ref[...]) + _dot(onehot, table2)
    u_ref[...] = _rowmask(i, u)


def _ln(t, g, b):
    mu = jnp.mean(t, axis=-1, keepdims=True)
    var = jnp.mean((t - mu) ** 2, axis=-1, keepdims=True)
    return (t - mu) / jnp.sqrt(var + 1e-5) * g + b


def _conv_body(u_ref, a_ref, b1_ref, w2_ref, b2_ref, g_ref, bb_ref, w1n_ref,
               o_ref):
    i = pl.program_id(0)
    z = jnp.maximum(u_ref[...] + a_ref[0] + a_ref[1] + b1_ref[...], 0.0)
    t = _dot(z, w2_ref[...]) + b2_ref[...]
    h = jnp.maximum(_ln(t, g_ref[...], bb_ref[...]), 0.0)
    o_ref[...] = _rowmask(i, _dot(h, w1n_ref[...]))


def _conv_gate_body(u_ref, a_ref, batch_ref, b1_ref, w2_ref, b2_ref, g_ref, bb_ref,
                    gw1_ref, gb1_ref, gw2_ref, gb2_ref,
                    h_out_ref, gate_out_ref, gmax_out_ref, gmax_acc):
    i = pl.program_id(0)
    nb = pl.num_programs(0)
    z = jnp.maximum(u_ref[...] + a_ref[0] + a_ref[1] + b1_ref[...], 0.0)
    t = _dot(z, w2_ref[...]) + b2_ref[...]
    h = jnp.maximum(_ln(t, g_ref[...], bb_ref[...]), 0.0)
    h = _rowmask(i, h)
    h_out_ref[...] = h
    g1 = jnp.maximum(_dot(h, gw1_ref[...]) + gb1_ref[...], 0.0)
    gate = _rowmask(i, _dot(g1, gw2_ref[...]) + gb2_ref[...])
    gate_out_ref[...] = gate
    m = batch_ref[...] == lax.broadcasted_iota(jnp.int32, (_R, _NG), 1)
    smax = jnp.max(jnp.where(m, gate, -1e30), axis=0, keepdims=True)

    @pl.when(i == 0)
    def _():
        gmax_acc[...] = jnp.full((8, _NG), -1e30, jnp.float32)

    gmax_acc[...] = jnp.maximum(gmax_acc[...], jnp.broadcast_to(smax, (8, _NG)))

    @pl.when(i == nb - 1)
    def _():
        gmax_out_ref[...] = gmax_acc[...]


def _pool_body(h_ref, gate_ref, gmax_ref, batch_ref, cw1_ref, cb1_ref, cw2_ref,
               cb2_ref, out_ref, num_acc, den_acc):
    i = pl.program_id(0)
    nb = pl.num_programs(0)

    @pl.when(i == 0)
    def _():
        num_acc[...] = jnp.zeros_like(num_acc)
        den_acc[...] = jnp.zeros_like(den_acc)

    m = (batch_ref[...] == lax.broadcasted_iota(jnp.int32, (_R, _NG), 1)
         ).astype(jnp.float32)
    gmax_i = jnp.sum(m * gmax_ref[0:1, :], axis=1, keepdims=True)
    alpha = jnp.exp(gate_ref[...] - gmax_i)
    wh = alpha * h_ref[...]
    cdims = (((0,), (0,)), ((), ()))
    num_acc[...] += lax.dot_general(m, wh, cdims, precision=_PREC,
                                    preferred_element_type=jnp.float32)
    den_acc[...] += lax.dot_general(m, alpha, cdims, precision=_PREC,
                                    preferred_element_type=jnp.float32)

    @pl.when(i == nb - 1)
    def _():
        pooled = num_acc[...] / (den_acc[...] + 1e-16)
        p1 = jnp.maximum(_dot(pooled, cw1_ref[...]) + cb1_ref[...], 0.0)
        out_ref[...] = _dot(p1, cw2_ref[...]) + cb2_ref[...]


def _row_spec(shape):
    nd = len(shape)
    return pl.BlockSpec(shape, lambda i: (i,) + (0,) * (nd - 1))


def _full_spec(shape):
    nd = len(shape)
    return pl.BlockSpec(shape, lambda i: (0,) * nd)


def kernel(x, edge_index, batch, emb, conv0_W1, conv0_b1, conv0_W2, conv0_b2,
           conv1_W1, conv1_b1, conv1_W2, conv1_b2, ln0_g, ln0_b, ln1_g, ln1_b,
           gate_W1, gate_b1, gate_W2, gate_b2, cls_W1, cls_b1, cls_W2, cls_b2):
    f32 = jnp.float32
    xp = jnp.pad(x, ((0, _NPAD - _N), (0, 0)))
    batch_p = jnp.pad(batch, (0, _NPAD - _N), constant_values=_NG).reshape(_NPAD, 1)
    pad_idx = _N + (jnp.arange(_EPAD - _E, dtype=jnp.int32) % (_NPAD - _N))
    src3 = jnp.concatenate([edge_index[0], pad_idx]).reshape(_NC, _NS, _CH, _CHUNK)
    dst3 = jnp.concatenate([edge_index[1], pad_idx]).reshape(_NC, _NS, _CH, _CHUNK)
    zrows = jnp.zeros((_ROWS_PER_SUB, _D), f32)
    w1a = jnp.pad(conv0_W1[:_D - 1], ((1, 0), (0, 0)))
    w1b = conv0_W1[_D - 1:]
    r2 = lambda v: v.reshape(1, -1)

    u0 = pl.pallas_call(
        _proj_body,
        grid=(_NB,),
        in_specs=[_row_spec((_R, _D)), _full_spec((_NT, 16)),
                  _full_spec((_D, _D)), _full_spec((16, _D))],
        out_specs=_row_spec((_R, _D)),
        out_shape=jax.ShapeDtypeStruct((_NPAD, _D), f32),
    )(xp, emb, w1a, w1b)

    agg0 = _sc_scatter_add(u0, src3, dst3, zrows)

    u1 = pl.pallas_call(
        _conv_body,
        grid=(_NB,),
        in_specs=[_row_spec((_R, _D)),
                  pl.BlockSpec((_NC, _R, _D), lambda i: (0, i, 0)),
                  _full_spec((1, _D)), _full_spec((_D, _D)), _full_spec((1, _D)),
                  _full_spec((1, _D)), _full_spec((1, _D)), _full_spec((_D, _D))],
        out_specs=_row_spec((_R, _D)),
        out_shape=jax.ShapeDtypeStruct((_NPAD, _D), f32),
    )(u0, agg0, r2(conv0_b1), conv0_W2, r2(conv0_b2), r2(ln0_g), r2(ln0_b),
      conv1_W1)

    agg1 = _sc_scatter_add(u1, src3, dst3, zrows)

    h2, gate, gmax = pl.pallas_call(
        _conv_gate_body,
        grid=(_NB,),
        in_specs=[_row_spec((_R, _D)),
                  pl.BlockSpec((_NC, _R, _D), lambda i: (0, i, 0)),
                  _row_spec((_R, 1)),
                  _full_spec((1, _D)), _full_spec((_D, _D)), _full_spec((1, _D)),
                  _full_spec((1, _D)), _full_spec((1, _D)),
                  _full_spec((_D, _D)), _full_spec((1, _D)),
                  _full_spec((_D, 1)), _full_spec((1, 1))],
        out_specs=[_row_spec((_R, _D)), _row_spec((_R, 1)), _full_spec((8, _NG))],
        out_shape=[jax.ShapeDtypeStruct((_NPAD, _D), f32),
                   jax.ShapeDtypeStruct((_NPAD, 1), f32),
                   jax.ShapeDtypeStruct((8, _NG), f32)],
        scratch_shapes=[pltpu.VMEM((8, _NG), f32)],
    )(u1, agg1, batch_p, r2(conv1_b1), conv1_W2, r2(conv1_b2), r2(ln1_g),
      r2(ln1_b), gate_W1, r2(gate_b1), gate_W2, gate_b2.reshape(1, 1))

    out = pl.pallas_call(
        _pool_body,
        grid=(_NB,),
        in_specs=[_row_spec((_R, _D)), _row_spec((_R, 1)), _full_spec((8, _NG)),
                  _row_spec((_R, 1)),
                  _full_spec((_D, _D)), _full_spec((1, _D)),
                  _full_spec((_D, 2)), _full_spec((1, 2))],
        out_specs=_full_spec((_NG, 2)),
        out_shape=jax.ShapeDtypeStruct((_NG, 2), f32),
        scratch_shapes=[pltpu.VMEM((_NG, _D), f32), pltpu.VMEM((_NG, 1), f32)],
    )(h2, gate, gmax, batch_p, cls_W1, r2(cls_b1), cls_W2, cls_b2.reshape(1, 2))

    return out


# speedup vs baseline: 3.6145x; 1.7999x over previous
import functools

import jax
import jax.numpy as jnp
from jax import lax
from jax.experimental import pallas as pl
from jax.experimental.pallas import tpu as pltpu
from jax.experimental.pallas import tpu_sc as plsc

_N = 10000
_NPAD = 10240
_D = 128
_NG = 64
_E = 320000
_NT = 400
_NC = 2
_NS = 16
_CHUNK = 128
_CH = 80
_EPAD = _NC * _NS * _CH * _CHUNK
_R = 1024
_NB = _NPAD // _R
_ROWS_PER_SUB = _NPAD // _NS
_PREC = lax.Precision.HIGHEST


def _sc_scatter_add(u, src3, dst3, zrows):
    mesh = plsc.VectorSubcoreMesh(core_axis_name="c", subcore_axis_name="s")

    @functools.partial(
        pl.kernel,
        mesh=mesh,
        out_type=jax.ShapeDtypeStruct((_NC, _NPAD, _D), jnp.float32),
        scratch_types=[
            pltpu.VMEM_SHARED((_NPAD, _D), jnp.float32),
            pltpu.VMEM((_CH, _CHUNK), jnp.int32),
            pltpu.VMEM((2, _CHUNK), jnp.int32),
            pltpu.VMEM((_CHUNK, _D), jnp.float32),
            pltpu.VMEM((_CHUNK, _D), jnp.float32),
            pltpu.SemaphoreType.DMA,
            pltpu.SemaphoreType.DMA,
            pltpu.SemaphoreType.DMA,
            pltpu.SemaphoreType.DMA,
        ],
    )
    def k(u_hbm, src_hbm, dst_hbm, z_hbm, out_hbm, agg_sh, sidx, didx,
          r0, r1, semA, semB, semDA, semDB):
        c = lax.axis_index("c")
        s = lax.axis_index("s")
        base = s * _ROWS_PER_SUB

        def fire(buf, p, semR, semD, kk):
            pltpu.async_copy(u_hbm.at[sidx.at[kk]], buf, semR)
            pltpu.async_copy(dst_hbm.at[c, s, kk], didx.at[p], semD)

        def drain_scatter(buf, p, semR, semD, kk):
            pltpu.make_async_copy(u_hbm.at[sidx.at[kk]], buf, semR).wait()
            pltpu.make_async_copy(dst_hbm.at[c, s, kk], didx.at[p], semD).wait()
            pltpu.sync_copy(buf, agg_sh.at[didx.at[p]], add=True)

        pltpu.sync_copy(src_hbm.at[c, s], sidx)
        fire(r0, 0, semA, semDA, 0)
        pltpu.sync_copy(z_hbm, agg_sh.at[pl.ds(base, _ROWS_PER_SUB)])
        plsc.subcore_barrier()

        @pl.loop(0, _CH // 2)
        def _(j):
            k0 = 2 * j
            k1 = 2 * j + 1
            fire(r1, 1, semB, semDB, k1)
            drain_scatter(r0, 0, semA, semDA, k0)

            @pl.when(k1 + 1 < _CH)
            def _():
                fire(r0, 0, semA, semDA, k1 + 1)

            drain_scatter(r1, 1, semB, semDB, k1)

        plsc.subcore_barrier()
        pltpu.sync_copy(agg_sh.at[pl.ds(base, _ROWS_PER_SUB)],
                        out_hbm.at[c, pl.ds(base, _ROWS_PER_SUB)])

    return k(u, src3, dst3, zrows)


def _rowmask(i, v):
    rid = i * _R + lax.broadcasted_iota(jnp.int32, (_R, 1), 0)
    return jnp.where(rid < _N, v, 0.0)


def _dot(a, b):
    return jnp.dot(a, b, precision=_PREC, preferred_element_type=jnp.float32)


def _proj_body(x_ref, emb_ref, w1a_ref, w1b_ref, u_ref):
    i = pl.program_id(0)
    x = x_ref[...]
    table2 = _dot(emb_ref[...], w1b_ref[...])
    types = x[:, 0:1].astype(jnp.int32)
    iota = lax.broadcasted_iota(jnp.int32, (_R, _NT), 1)
    onehot = (types == iota).astype(jnp.float32)
    u = _dot(x, w1a_ref[...]) + _dot(onehot, table2)
    u_ref[...] = _rowmask(i, u)


def _ln(t, g, b):
    mu = jnp.mean(t, axis=-1, keepdims=True)
    var = jnp.mean((t - mu) ** 2, axis=-1, keepdims=True)
    return (t - mu) / jnp.sqrt(var + 1e-5) * g + b


def _conv_body(u_ref, a_ref, b1_ref, w2_ref, b2_ref, g_ref, bb_ref, w1n_ref,
               o_ref):
    i = pl.program_id(0)
    z = jnp.maximum(u_ref[...] + a_ref[0] + a_ref[1] + b1_ref[...], 0.0)
    t = _dot(z, w2_ref[...]) + b2_ref[...]
    h = jnp.maximum(_ln(t, g_ref[...], bb_ref[...]), 0.0)
    o_ref[...] = _rowmask(i, _dot(h, w1n_ref[...]))


def _conv_gate_body(u_ref, a_ref, batch_ref, b1_ref, w2_ref, b2_ref, g_ref, bb_ref,
                    gw1_ref, gb1_ref, gw2_ref, gb2_ref,
                    h_out_ref, gate_out_ref, gmax_out_ref, gmax_acc):
    i = pl.program_id(0)
    nb = pl.num_programs(0)
    z = jnp.maximum(u_ref[...] + a_ref[0] + a_ref[1] + b1_ref[...], 0.0)
    t = _dot(z, w2_ref[...]) + b2_ref[...]
    h = jnp.maximum(_ln(t, g_ref[...], bb_ref[...]), 0.0)
    h = _rowmask(i, h)
    h_out_ref[...] = h
    g1 = jnp.maximum(_dot(h, gw1_ref[...]) + gb1_ref[...], 0.0)
    gate = _rowmask(i, _dot(g1, gw2_ref[...]) + gb2_ref[...])
    gate_out_ref[...] = gate
    m = batch_ref[...] == lax.broadcasted_iota(jnp.int32, (_R, _NG), 1)
    smax = jnp.max(jnp.where(m, gate, -1e30), axis=0, keepdims=True)

    @pl.when(i == 0)
    def _():
        gmax_acc[...] = jnp.full((8, _NG), -1e30, jnp.float32)

    gmax_acc[...] = jnp.maximum(gmax_acc[...], jnp.broadcast_to(smax, (8, _NG)))

    @pl.when(i == nb - 1)
    def _():
        gmax_out_ref[...] = gmax_acc[...]


def _pool_body(h_ref, gate_ref, gmax_ref, batch_ref, cw1_ref, cb1_ref, cw2_ref,
               cb2_ref, out_ref, num_acc, den_acc):
    i = pl.program_id(0)
    nb = pl.num_programs(0)

    @pl.when(i == 0)
    def _():
        num_acc[...] = jnp.zeros_like(num_acc)
        den_acc[...] = jnp.zeros_like(den_acc)

    m = (batch_ref[...] == lax.broadcasted_iota(jnp.int32, (_R, _NG), 1)
         ).astype(jnp.float32)
    gmax_i = jnp.sum(m * gmax_ref[0:1, :], axis=1, keepdims=True)
    alpha = jnp.exp(gate_ref[...] - gmax_i)
    wh = alpha * h_ref[...]
    cdims = (((0,), (0,)), ((), ()))
    num_acc[...] += lax.dot_general(m, wh, cdims, precision=_PREC,
                                    preferred_element_type=jnp.float32)
    den_acc[...] += lax.dot_general(m, alpha, cdims, precision=_PREC,
                                    preferred_element_type=jnp.float32)

    @pl.when(i == nb - 1)
    def _():
        pooled = num_acc[...] / (den_acc[...] + 1e-16)
        p1 = jnp.maximum(_dot(pooled, cw1_ref[...]) + cb1_ref[...], 0.0)
        out_ref[...] = _dot(p1, cw2_ref[...]) + cb2_ref[...]


def _row_spec(shape):
    nd = len(shape)
    return pl.BlockSpec(shape, lambda i: (i,) + (0,) * (nd - 1))


def _full_spec(shape):
    nd = len(shape)
    return pl.BlockSpec(shape, lambda i: (0,) * nd)


def kernel(x, edge_index, batch, emb, conv0_W1, conv0_b1, conv0_W2, conv0_b2,
           conv1_W1, conv1_b1, conv1_W2, conv1_b2, ln0_g, ln0_b, ln1_g, ln1_b,
           gate_W1, gate_b1, gate_W2, gate_b2, cls_W1, cls_b1, cls_W2, cls_b2):
    f32 = jnp.float32
    xp = jnp.pad(x, ((0, _NPAD - _N), (0, 0)))
    batch_p = jnp.pad(batch, (0, _NPAD - _N), constant_values=_NG).reshape(_NPAD, 1)
    pad_idx = _N + (jnp.arange(_EPAD - _E, dtype=jnp.int32) % (_NPAD - _N))
    src3 = jnp.concatenate([edge_index[0], pad_idx]).reshape(_NC, _NS, _CH, _CHUNK)
    dst3 = jnp.concatenate([edge_index[1], pad_idx]).reshape(_NC, _NS, _CH, _CHUNK)
    zrows = jnp.zeros((_ROWS_PER_SUB, _D), f32)
    w1a = jnp.pad(conv0_W1[:_D - 1], ((1, 0), (0, 0)))
    w1b = conv0_W1[_D - 1:]
    r2 = lambda v: v.reshape(1, -1)

    u0 = pl.pallas_call(
        _proj_body,
        grid=(_NB,),
        in_specs=[_row_spec((_R, _D)), _full_spec((_NT, 16)),
                  _full_spec((_D, _D)), _full_spec((16, _D))],
        out_specs=_row_spec((_R, _D)),
        out_shape=jax.ShapeDtypeStruct((_NPAD, _D), f32),
    )(xp, emb, w1a, w1b)

    agg0 = _sc_scatter_add(u0, src3, dst3, zrows)

    u1 = pl.pallas_call(
        _conv_body,
        grid=(_NB,),
        in_specs=[_row_spec((_R, _D)),
                  pl.BlockSpec((_NC, _R, _D), lambda i: (0, i, 0)),
                  _full_spec((1, _D)), _full_spec((_D, _D)), _full_spec((1, _D)),
                  _full_spec((1, _D)), _full_spec((1, _D)), _full_spec((_D, _D))],
        out_specs=_row_spec((_R, _D)),
        out_shape=jax.ShapeDtypeStruct((_NPAD, _D), f32),
    )(u0, agg0, r2(conv0_b1), conv0_W2, r2(conv0_b2), r2(ln0_g), r2(ln0_b),
      conv1_W1)

    agg1 = _sc_scatter_add(u1, src3, dst3, zrows)

    h2, gate, gmax = pl.pallas_call(
        _conv_gate_body,
        grid=(_NB,),
        in_specs=[_row_spec((_R, _D)),
                  pl.BlockSpec((_NC, _R, _D), lambda i: (0, i, 0)),
                  _row_spec((_R, 1)),
                  _full_spec((1, _D)), _full_spec((_D, _D)), _full_spec((1, _D)),
                  _full_spec((1, _D)), _full_spec((1, _D)),
                  _full_spec((_D, _D)), _full_spec((1, _D)),
                  _full_spec((_D, 1)), _full_spec((1, 1))],
        out_specs=[_row_spec((_R, _D)), _row_spec((_R, 1)), _full_spec((8, _NG))],
        out_shape=[jax.ShapeDtypeStruct((_NPAD, _D), f32),
                   jax.ShapeDtypeStruct((_NPAD, 1), f32),
                   jax.ShapeDtypeStruct((8, _NG), f32)],
        scratch_shapes=[pltpu.VMEM((8, _NG), f32)],
    )(u1, agg1, batch_p, r2(conv1_b1), conv1_W2, r2(conv1_b2), r2(ln1_g),
      r2(ln1_b), gate_W1, r2(gate_b1), gate_W2, gate_b2.reshape(1, 1))

    out = pl.pallas_call(
        _pool_body,
        grid=(_NB,),
        in_specs=[_row_spec((_R, _D)), _row_spec((_R, 1)), _full_spec((8, _NG)),
                  _row_spec((_R, 1)),
                  _full_spec((_D, _D)), _full_spec((1, _D)),
                  _full_spec((_D, 2)), _full_spec((1, 2))],
        out_specs=_full_spec((_NG, 2)),
        out_shape=jax.ShapeDtypeStruct((_NG, 2), f32),
        scratch_shapes=[pltpu.VMEM((_NG, _D), f32), pltpu.VMEM((_NG, 1), f32)],
    )(h2, gate, gmax, batch_p, cls_W1, r2(cls_b1), cls_W2, cls_b2.reshape(1, 2))

    return out
